# Initial kernel scaffold; baseline (speedup 1.0000x reference)
import jax
import jax.numpy as jnp
from jax.experimental import pallas as pl


def kernel(x, edge_index, deg, W_self, b_self, W_neigh, b_neigh):
    raise NotImplementedError("write your pallas kernel here")


# speedup vs baseline: 8.8668x; 8.8668x over previous
import functools

import jax
import jax.numpy as jnp
from jax import lax
from jax.experimental import pallas as pl
from jax.experimental.pallas import tpu as pltpu
from jax.experimental.pallas import tpu_sc as plsc

NC = 2
NS = 16
NW = NC * NS
LANES = 16
CHUNK = 80
ZROWS = CHUNK


def _pad_rows(n_nodes: int, d: int) -> int:
    unit = NS * ZROWS
    per = -(-n_nodes // unit) * unit
    assert per % d == 0
    return per


@functools.cache
def _segsum_fn(n_nodes: int, d: int, n_edges: int):
    epw = n_edges // NW
    nchunk = epw // CHUNK
    assert nchunk % 2 == 1 and nchunk >= 3
    n_pad = _pad_rows(n_nodes, d)
    rpt = n_pad // NS
    crows = n_pad // d
    assert d & (d - 1) == 0
    dshift = d.bit_length() - 1

    mesh = plsc.VectorSubcoreMesh(
        core_axis_name="c", subcore_axis_name="s",
        num_cores=NC, num_subcores=NS)

    @functools.partial(
        pl.kernel,
        compiler_params=pltpu.CompilerParams(needs_layout_passes=False),
        out_type=(
            jax.ShapeDtypeStruct((NC, n_pad, d), jnp.float32),
            jax.ShapeDtypeStruct((NC, crows, d), jnp.int32),
        ),
        mesh=mesh,
        scratch_types=[
            pltpu.VMEM_SHARED((n_pad, d), jnp.float32),
            pltpu.VMEM_SHARED((crows, d), jnp.int32),
            pltpu.VMEM((CHUNK,), jnp.int32),
            pltpu.VMEM((CHUNK,), jnp.int32),
            pltpu.VMEM((nchunk, CHUNK), jnp.int32),
            pltpu.VMEM((CHUNK, d), jnp.float32),
            pltpu.VMEM((CHUNK, d), jnp.float32),
            pltpu.VMEM((crows, d), jnp.int32),
            pltpu.VMEM((crows,), jnp.int32),
            pltpu.SemaphoreType.DMA,
            pltpu.SemaphoreType.DMA,
            pltpu.SemaphoreType.DMA,
            pltpu.SemaphoreType.DMA,
            pltpu.SemaphoreType.DMA,
            pltpu.SemaphoreType.DMA,
        ],
    )
    def segsum(x_hbm, src_hbm, dst_hbm, feat_hbm, cnt_hbm,
               acc, cacc, idx_s0, idx_s1, idx_d, rows0, rows1, cnt, idx_id,
               si0, si1, sg0, sg1, ss0, ss1):
        c = lax.axis_index("c")
        s = lax.axis_index("s")
        wid = s * NC + c
        idx_s = (idx_s0, idx_s1)
        rows = (rows0, rows1)
        si = (si0, si1)
        sg = (sg0, sg1)
        ss = (ss0, ss1)

        idx_cp_d = pltpu.async_copy(dst_hbm.at[wid], idx_d, ss0)

        def zrow(i, carry):
            for j in range(d // LANES):
                rows0[i, pl.ds(j * LANES, LANES)] = jnp.zeros(
                    (LANES,), jnp.float32)
            return carry
        lax.fori_loop(0, ZROWS, zrow, 0)

        def crow(i, carry):
            for j in range(d // LANES):
                cnt[i, pl.ds(j * LANES, LANES)] = jnp.zeros(
                    (LANES,), jnp.int32)
            return carry
        lax.fori_loop(0, crows, crow, 0)

        for j in range(crows // LANES):
            idx_id[pl.ds(j * LANES, LANES)] = (
                lax.iota(jnp.int32, LANES) + j * LANES)

        for k in range(rpt // ZROWS):
            pltpu.sync_copy(rows0, acc.at[pl.ds(s * rpt + k * ZROWS, ZROWS)])
        @pl.when(s == 0)
        def _clear_cacc():
            pltpu.sync_copy(cnt, cacc)
        idx_cp_d.wait()
        plsc.subcore_barrier()

        ones16 = jnp.ones((LANES,), jnp.int32)
        last = nchunk - 1

        def start_idx(j, p):
            jc = jnp.minimum(j, last)
            pltpu.async_copy(src_hbm.at[wid, jc], idx_s[p], si[p])

        def wait_idx(p):
            pltpu.make_async_copy(src_hbm.at[0, 0], idx_s[p], si[p]).wait()

        def start_gather(j, p):
            del j
            pltpu.async_copy(x_hbm.at[idx_s[p]], rows[p], sg[p])

        def wait_gather(p):
            pltpu.make_async_copy(x_hbm.at[idx_s[p]], rows[p],
                                  sg[p]).wait()

        def start_scatter(j, p):
            pltpu.async_copy(rows[p], acc.at[idx_d.at[j]], ss[p], add=True)

        def wait_scatter(p):
            pltpu.make_async_copy(rows[p], acc.at[idx_d.at[0]],
                                  ss[p]).wait()

        def hist(j):
            for v in range(CHUNK // LANES):
                dv = idx_d[j, pl.ds(v * LANES, LANES)]
                plsc.addupdate_scatter(
                    cnt,
                    [lax.shift_right_logical(dv, dshift),
                     lax.bitwise_and(dv, d - 1)],
                    ones16)

        start_idx(0, 0)
        start_idx(1, 1)
        wait_idx(0)
        start_gather(0, 0)
        wait_gather(0)
        start_scatter(0, 0)
        start_idx(2, 0)
        hist(0)
        wait_idx(1)
        start_gather(1, 1)
        wait_gather(1)
        start_scatter(1, 1)
        start_idx(3, 1)
        hist(1)
        wait_scatter(0)
        wait_idx(0)
        start_gather(2, 0)

        def body(t, carry):
            j0 = 2 * t
            wait_gather(0)
            start_scatter(j0, 0)
            start_idx(j0 + 2, 0)
            hist(j0)
            wait_scatter(1)
            wait_idx(1)
            start_gather(j0 + 1, 1)
            wait_gather(1)
            start_scatter(j0 + 1, 1)
            start_idx(j0 + 3, 1)
            hist(j0 + 1)
            wait_scatter(0)
            wait_idx(0)
            start_gather(j0 + 2, 0)
            return carry
        lax.fori_loop(1, (nchunk - 1) // 2, body, 0)

        wait_gather(0)
        start_scatter(last, 0)
        hist(last)
        wait_idx(1)
        wait_scatter(1)
        wait_scatter(0)

        pltpu.sync_copy(cnt, cacc.at[idx_id], add=True)
        plsc.subcore_barrier()

        pltpu.sync_copy(acc.at[pl.ds(s * rpt, rpt)],
                        feat_hbm.at[c, pl.ds(s * rpt, rpt)])
        @pl.when(s == 0)
        def _dump_cnt():
            pltpu.sync_copy(cacc, cnt_hbm.at[c])

    return segsum


@functools.cache
def _dense_fn(n_nodes: int, d_in: int, d_out: int):
    blk = 400
    grid = n_nodes // blk

    def body(x_ref, p_ref, cnt_ref, deg_ref, wst_ref, bs_ref, wnt_ref,
             bn_ref, o_ref):
        ns = p_ref[0] + p_ref[1]
        cnt = (cnt_ref[0] + cnt_ref[1]).astype(jnp.float32)
        agg = (jnp.dot(ns, wnt_ref[...],
                       preferred_element_type=jnp.float32)
               + cnt * bn_ref[...])
        denom = jnp.maximum(deg_ref[...], 1.0)
        z = (jnp.dot(x_ref[...], wst_ref[...],
                     preferred_element_type=jnp.float32)
             + bs_ref[...] + agg / denom)
        o_ref[...] = jnp.where(z >= 0.0, z, 0.1 * z)

    return pl.pallas_call(
        body,
        grid=(grid,),
        in_specs=[
            pl.BlockSpec((blk, d_in), lambda i: (i, 0)),
            pl.BlockSpec((NC, blk, d_in), lambda i: (0, i, 0)),
            pl.BlockSpec((NC, blk, 1), lambda i: (0, i, 0)),
            pl.BlockSpec((blk, 1), lambda i: (i, 0)),
            pl.BlockSpec((d_in, d_out), lambda i: (0, 0)),
            pl.BlockSpec((1, d_out), lambda i: (0, 0)),
            pl.BlockSpec((d_in, d_out), lambda i: (0, 0)),
            pl.BlockSpec((1, d_out), lambda i: (0, 0)),
        ],
        out_specs=pl.BlockSpec((blk, d_out), lambda i: (i, 0)),
        out_shape=jax.ShapeDtypeStruct((n_nodes, d_out), jnp.float32),
    )


def kernel(x, edge_index, deg, W_self, b_self, W_neigh, b_neigh):
    b, n_nodes, d_in = x.shape
    d_out = W_neigh.shape[0]
    n_edges = edge_index.shape[1]
    epw = n_edges // NW
    nchunk = epw // CHUNK

    src = edge_index[0].astype(jnp.int32).reshape(NW, nchunk, CHUNK)
    dst = edge_index[1].astype(jnp.int32).reshape(NW, nchunk, CHUNK)
    deg_f = jnp.asarray(deg).astype(jnp.float32).reshape(n_nodes, 1)
    wst = W_self.astype(jnp.float32).T
    wnt = W_neigh.astype(jnp.float32).T
    bs = b_self.astype(jnp.float32).reshape(1, d_out)
    bn = b_neigh.astype(jnp.float32).reshape(1, d_out)

    segsum = _segsum_fn(n_nodes, d_in, n_edges)
    dense = _dense_fn(n_nodes, d_in, d_out)

    outs = []
    for bi in range(b):
        xb = x[bi].astype(jnp.float32)
        feat, cnt = segsum(xb, src, dst)
        cnt_n = cnt.reshape(NC, -1)[:, :n_nodes, None]
        outs.append(dense(xb, feat, cnt_n, deg_f, wst, bs, wnt, bn))
    return jnp.stack(outs, axis=0).astype(x.dtype)
